# baseline (speedup 1.0000x reference)
import functools

import jax
import jax.numpy as jnp
from jax import lax
from jax.experimental import pallas as pl
from jax.experimental.pallas import tpu as pltpu
from jax.experimental.pallas import tpu_sc as plsc

NUM_FIELDS = 26
VOCAB = 100000
EMB_DIM = 32
BATCH = 16384

_info = plsc.get_sparse_core_info()
_NC, _NS, _L = _info.num_cores, _info.num_subcores, _info.num_lanes
_NW = _NC * _NS

TOTAL_ROWS = BATCH * NUM_FIELDS
ROWS_PER_W = TOTAL_ROWS // _NW
G = 128
CHUNK = 13 * G
NGATHER = CHUNK // G
NCHUNK = ROWS_PER_W // CHUNK

_mesh = plsc.VectorSubcoreMesh(core_axis_name="c", subcore_axis_name="s")


@functools.partial(
    pl.kernel,
    mesh=_mesh,
    compiler_params=pltpu.CompilerParams(use_tc_tiling_on_sc=False),
    out_type=jax.ShapeDtypeStruct((TOTAL_ROWS, EMB_DIM), jnp.float32),
    scratch_types=[
        pltpu.VMEM((CHUNK,), jnp.int32),
        pltpu.VMEM((CHUNK,), jnp.int32),
        pltpu.VMEM((NGATHER, G), jnp.int32),
        pltpu.VMEM((CHUNK, EMB_DIM), jnp.float32),
        pltpu.SemaphoreType.DMA,
    ],
)
def _emb_lookup(x_hbm, offs_hbm, w_hbm, out_hbm, raw_v, offs_v, idx_v, rows_v, sem):
    wid = lax.axis_index("s") * _NC + lax.axis_index("c")
    base = wid * ROWS_PER_W
    pltpu.sync_copy(offs_hbm, offs_v)

    def chunk_body(c, carry):
        row0 = base + c * CHUNK
        pltpu.sync_copy(x_hbm.at[pl.ds(row0, CHUNK)], raw_v)

        for g in range(NGATHER):
            def add_body(j, carry2):
                s = pl.ds(g * G + j * _L, _L)
                idx_v[g, pl.ds(j * _L, _L)] = raw_v[s] + offs_v[s]
                return carry2
            lax.fori_loop(0, G // _L, add_body, 0, unroll=True)

        copies = []
        for g in range(NGATHER):
            copies.append(
                pltpu.async_copy(
                    w_hbm.at[idx_v.at[g]], rows_v.at[pl.ds(g * G, G)], sem
                )
            )
        for cp in copies:
            cp.wait()

        pltpu.sync_copy(rows_v, out_hbm.at[pl.ds(row0, CHUNK)])
        return carry

    lax.fori_loop(0, NCHUNK, chunk_body, 0)


def kernel(x, W):
    x_flat = x.astype(jnp.int32).reshape(TOTAL_ROWS)
    offs = jnp.tile(
        jnp.arange(NUM_FIELDS, dtype=jnp.int32) * VOCAB, CHUNK // NUM_FIELDS
    )
    w_flat = W.reshape(NUM_FIELDS * VOCAB, EMB_DIM)
    out = _emb_lookup(x_flat, offs, w_flat)
    return out.reshape(BATCH, NUM_FIELDS * EMB_DIM)


# speedup vs baseline: 3.1973x; 3.1973x over previous
import functools

import jax
import jax.numpy as jnp
from jax import lax
from jax.experimental import pallas as pl
from jax.experimental.pallas import tpu as pltpu
from jax.experimental.pallas import tpu_sc as plsc

NUM_FIELDS = 26
VOCAB = 100000
EMB_DIM = 32
BATCH = 16384

_info = plsc.get_sparse_core_info()
_NC, _NS, _L = _info.num_cores, _info.num_subcores, _info.num_lanes
_NW = _NC * _NS

PAIRS = NUM_FIELDS * EMB_DIM
PAIRS_PER_W = PAIRS // _NW
CB = 4096
NCB = BATCH // CB

_mesh = plsc.VectorSubcoreMesh(core_axis_name="c", subcore_axis_name="s")


@functools.partial(
    pl.kernel,
    mesh=_mesh,
    compiler_params=pltpu.CompilerParams(needs_layout_passes=False),
    out_type=jax.ShapeDtypeStruct((PAIRS, BATCH), jnp.float32),
    scratch_types=[
        pltpu.VMEM((VOCAB,), jnp.float32),
        pltpu.VMEM((CB,), jnp.int32),
        pltpu.VMEM((CB,), jnp.float32),
    ],
)
def _emb_lookup(wT_hbm, xT_hbm, out_hbm, vocab_v, x_v, out_v):
    wid = lax.axis_index("s") * _NC + lax.axis_index("c")

    def pair_body(i, carry):
        p = wid * PAIRS_PER_W + i
        f = p // EMB_DIM
        d = p % EMB_DIM
        pltpu.sync_copy(wT_hbm.at[f, d], vocab_v)

        def cb_body(cb, c2):
            pltpu.sync_copy(xT_hbm.at[f, pl.ds(cb * CB, CB)], x_v)

            def g_body(j, c3):
                s = pl.ds(j * _L, _L)
                out_v[s] = plsc.load_gather(vocab_v, [x_v[s]])
                return c3

            lax.fori_loop(0, CB // _L, g_body, 0)
            pltpu.sync_copy(out_v, out_hbm.at[p, pl.ds(cb * CB, CB)])
            return c2

        lax.fori_loop(0, NCB, cb_body, 0)
        return carry

    lax.fori_loop(0, PAIRS_PER_W, pair_body, 0)


def kernel(x, W):
    wT = jnp.transpose(W, (0, 2, 1))
    xT = x.astype(jnp.int32).T
    out = _emb_lookup(wT, xT)
    return out.T
